# baseline (device time: 173976 ns/iter reference)
import jax
import jax.numpy as jnp
from jax import lax
from jax.experimental import pallas as pl
from jax.experimental.pallas import tpu as pltpu

N_DEV = 4
SQ = 2048
D_MODEL = 1024
H_PER = 8
DH = 128
BLK = 64
N_RES = 4
BLKS_PER_RES = SQ // BLK // N_RES
GROUP = BLKS_PER_RES * BLK
SCALE = 0.08838834764831843
CHUNK = SQ // N_DEV
CBLKS = CHUNK // BLK // N_RES
SCALE_BF = jnp.bfloat16(SCALE)


def _fused_body(x_ref, wq_ref, k_ref, v_ref, wo_ref, out_ref,
                w_ref, rs_r, rs_l, send_sems, recv_sems):
    my_pos = lax.axis_index("i")
    left = lax.rem(my_pos + N_DEV - 1, N_DEV)
    right = lax.rem(my_pos + 1, N_DEV)
    HALF = D_MODEL // 2

    def compute_chunk(coff):
        xc = x_ref[pl.ds(coff, CHUNK), :].astype(jnp.bfloat16)
        out_ref[pl.ds(coff, CHUNK), :] = jnp.zeros(
            (CHUNK, D_MODEL), jnp.float32)

        def head_body(h, carry):
            wq_h = wq_ref[:, pl.ds(h * DH, DH)].astype(jnp.bfloat16)
            q_c = jnp.dot(xc, wq_h, preferred_element_type=jnp.float32)
            qg = q_c.astype(jnp.bfloat16).reshape(CBLKS, N_RES, BLK, DH)
            k_h = k_ref[:, pl.ds(h, 1), :].reshape(SQ, DH).astype(
                jnp.bfloat16)
            v_h = v_ref[:, pl.ds(h, 1), :].reshape(SQ, DH).astype(
                jnp.bfloat16)
            kg = k_h.reshape(BLKS_PER_RES, N_RES, BLK, DH)
            vg = v_h.reshape(BLKS_PER_RES, N_RES, BLK, DH)
            parts = []
            for r in range(N_RES):
                qr = qg[:, r].reshape(CBLKS * BLK, DH)
                kr = kg[:, r].reshape(GROUP, DH)
                vr = vg[:, r].reshape(GROUP, DH)
                s = jnp.dot(qr, kr.T,
                            preferred_element_type=jnp.float32) * SCALE
                e = jnp.exp(s)
                ssum = jnp.sum(e, axis=-1, keepdims=True)
                pv = jnp.dot(e.astype(jnp.bfloat16), vr,
                             preferred_element_type=jnp.float32)
                parts.append((pv / ssum).reshape(CBLKS, BLK, DH))
            ctx = (jnp.stack(parts, axis=1)
                   .reshape(CHUNK, DH).astype(jnp.bfloat16))
            wo_h = wo_ref[pl.ds(h * DH, DH), :].astype(jnp.bfloat16)
            out_ref[pl.ds(coff, CHUNK), :] = (
                out_ref[pl.ds(coff, CHUNK), :]
                + jnp.dot(ctx, wo_h, preferred_element_type=jnp.float32)
            )
            return carry

        lax.fori_loop(0, H_PER, head_body, 0)
        w_ref[pl.ds(coff, CHUNK), :] = (
            out_ref[pl.ds(coff, CHUNK), :].astype(jnp.bfloat16))

    def off(dk):
        return lax.rem(my_pos + dk, N_DEV) * CHUNK

    compute_chunk(off(3))
    compute_chunk(off(1))

    barrier_sem = pltpu.get_barrier_semaphore()
    for nbr in (left, right):
        pl.semaphore_signal(
            barrier_sem, inc=1,
            device_id=(nbr,), device_id_type=pl.DeviceIdType.MESH,
        )
    pl.semaphore_wait(barrier_sem, 2)

    def rs_step(s, sc_r, rc_r, sc_l, rc_l, during=None):
        rdma_r = pltpu.make_async_remote_copy(
            src_ref=w_ref.at[pl.ds(sc_r * CHUNK, CHUNK), 0:HALF],
            dst_ref=rs_r.at[s],
            send_sem=send_sems.at[s],
            recv_sem=recv_sems.at[s],
            device_id=(right,),
            device_id_type=pl.DeviceIdType.MESH,
        )
        rdma_l = pltpu.make_async_remote_copy(
            src_ref=w_ref.at[pl.ds(sc_l * CHUNK, CHUNK), HALF:D_MODEL],
            dst_ref=rs_l.at[s],
            send_sem=send_sems.at[6 + s],
            recv_sem=recv_sems.at[6 + s],
            device_id=(left,),
            device_id_type=pl.DeviceIdType.MESH,
        )
        rdma_r.start()
        rdma_l.start()
        if during is not None:
            during()
        rdma_r.wait()
        rdma_l.wait()
        w_ref[pl.ds(rc_r * CHUNK, CHUNK), 0:HALF] = (
            w_ref[pl.ds(rc_r * CHUNK, CHUNK), 0:HALF].astype(jnp.float32)
            + rs_r[s].astype(jnp.float32)
        ).astype(jnp.bfloat16)
        w_ref[pl.ds(rc_l * CHUNK, CHUNK), HALF:D_MODEL] = (
            w_ref[pl.ds(rc_l * CHUNK, CHUNK), HALF:D_MODEL]
            .astype(jnp.float32)
            + rs_l[s].astype(jnp.float32)
        ).astype(jnp.bfloat16)

    rs_step(0, lax.rem(my_pos + 3, N_DEV), lax.rem(my_pos + 2, N_DEV),
            lax.rem(my_pos + 1, N_DEV), lax.rem(my_pos + 2, N_DEV),
            during=lambda: compute_chunk(off(2)))
    rs_step(1, lax.rem(my_pos + 2, N_DEV), lax.rem(my_pos + 1, N_DEV),
            lax.rem(my_pos + 2, N_DEV), lax.rem(my_pos + 3, N_DEV),
            during=lambda: compute_chunk(off(0)))
    rs_step(2, lax.rem(my_pos + 1, N_DEV), lax.rem(my_pos + 0, N_DEV),
            lax.rem(my_pos + 3, N_DEV), lax.rem(my_pos + 0, N_DEV))

    for t in range(N_DEV - 1):
        gc_r = lax.rem(my_pos + N_DEV - t, N_DEV)
        gc_l = lax.rem(my_pos + t, N_DEV)
        rdma_r = pltpu.make_async_remote_copy(
            src_ref=w_ref.at[pl.ds(gc_r * CHUNK, CHUNK), 0:HALF],
            dst_ref=w_ref.at[pl.ds(gc_r * CHUNK, CHUNK), 0:HALF],
            send_sem=send_sems.at[N_DEV - 1 + t],
            recv_sem=recv_sems.at[N_DEV - 1 + t],
            device_id=(right,),
            device_id_type=pl.DeviceIdType.MESH,
        )
        rdma_l = pltpu.make_async_remote_copy(
            src_ref=w_ref.at[pl.ds(gc_l * CHUNK, CHUNK), HALF:D_MODEL],
            dst_ref=w_ref.at[pl.ds(gc_l * CHUNK, CHUNK), HALF:D_MODEL],
            send_sem=send_sems.at[6 + N_DEV - 1 + t],
            recv_sem=recv_sems.at[6 + N_DEV - 1 + t],
            device_id=(left,),
            device_id_type=pl.DeviceIdType.MESH,
        )
        rdma_r.start()
        rdma_l.start()
        rdma_r.wait()
        rdma_l.wait()
    out_ref[...] = w_ref[...].astype(jnp.float32)


def kernel(x, Wq, K_ext, V_ext, Wo):
    my = lax.axis_index("i")
    Wq_my = lax.dynamic_slice(Wq, (0, my * H_PER * DH), (D_MODEL, H_PER * DH))
    Wo_my = lax.dynamic_slice(Wo, (my * H_PER * DH, 0), (H_PER * DH, D_MODEL))

    out = pl.pallas_call(
        _fused_body,
        out_shape=jax.ShapeDtypeStruct((SQ, D_MODEL), jnp.float32),
        in_specs=[pl.BlockSpec(memory_space=pltpu.VMEM)] * 5,
        out_specs=pl.BlockSpec(memory_space=pltpu.VMEM),
        scratch_shapes=[
            pltpu.VMEM((SQ, D_MODEL), jnp.bfloat16),
            pltpu.VMEM((N_DEV - 1, CHUNK, D_MODEL // 2), jnp.bfloat16),
            pltpu.VMEM((N_DEV - 1, CHUNK, D_MODEL // 2), jnp.bfloat16),
            pltpu.SemaphoreType.DMA((12,)),
            pltpu.SemaphoreType.DMA((12,)),
        ],
        compiler_params=pltpu.CompilerParams(
            collective_id=0,
            vmem_limit_bytes=60 * 1024 * 1024,
        ),
    )(x[0], Wq_my, K_ext[0], V_ext[0], Wo_my)
    return out[None]


# device time: 132250 ns/iter; 1.3155x vs baseline; 1.3155x over previous
import jax
import jax.numpy as jnp
from jax import lax
from jax.experimental import pallas as pl
from jax.experimental.pallas import tpu as pltpu

N_DEV = 4
SQ = 2048
D_MODEL = 1024
H_PER = 8
DH = 128
BLK = 64
N_RES = 4
BLKS_PER_RES = SQ // BLK // N_RES
GROUP = BLKS_PER_RES * BLK
SCALE = 0.08838834764831843


def _compute_body(x_ref, wq_ref, k_ref, v_ref, wo_ref, out_ref, acc_ref):
    xv = x_ref[...].astype(jnp.bfloat16)
    acc_ref[...] = jnp.zeros((SQ, D_MODEL), jnp.float32)
    for h in range(H_PER):
        wq_h = wq_ref[:, h * DH:(h + 1) * DH].astype(jnp.bfloat16)
        q_h = jnp.dot(xv, wq_h,
                      preferred_element_type=jnp.float32)
        k_h = k_ref[:, h, :].astype(jnp.bfloat16)
        v_h = v_ref[:, h, :].astype(jnp.bfloat16)
        qg = q_h.astype(jnp.bfloat16).reshape(BLKS_PER_RES, N_RES, BLK, DH)
        kg = k_h.reshape(BLKS_PER_RES, N_RES, BLK, DH)
        vg = v_h.reshape(BLKS_PER_RES, N_RES, BLK, DH)
        parts = []
        for r in range(N_RES):
            qr = qg[:, r].reshape(GROUP, DH)
            kr = kg[:, r].reshape(GROUP, DH)
            vr = vg[:, r].reshape(GROUP, DH)
            s = jnp.dot(qr, kr.T, preferred_element_type=jnp.float32) * SCALE
            e = jnp.exp(s)
            ssum = jnp.sum(e, axis=-1, keepdims=True)
            pv = jnp.dot(e.astype(jnp.bfloat16), vr,
                         preferred_element_type=jnp.float32)
            parts.append((pv / ssum).reshape(BLKS_PER_RES, BLK, DH))
        ctx_h = jnp.stack(parts, axis=1).reshape(SQ, DH).astype(jnp.bfloat16)
        wo_h = wo_ref[h * DH:(h + 1) * DH, :].astype(jnp.bfloat16)
        acc_ref[...] = acc_ref[...] + jnp.dot(
            ctx_h, wo_h, preferred_element_type=jnp.float32)
    out_ref[...] = acc_ref[...].astype(jnp.bfloat16)


CHUNK = SQ // N_DEV
HALF = D_MODEL // 2


def _allreduce_body(p_ref, out_ref, w_ref, rs_r, rs_l, send_sems, recv_sems):
    my_pos = lax.axis_index("i")
    left = lax.rem(my_pos + N_DEV - 1, N_DEV)
    right = lax.rem(my_pos + 1, N_DEV)

    barrier_sem = pltpu.get_barrier_semaphore()
    for nbr in (left, right):
        pl.semaphore_signal(
            barrier_sem, inc=1,
            device_id=(nbr,), device_id_type=pl.DeviceIdType.MESH,
        )
    pl.semaphore_wait(barrier_sem, 2)

    w_ref[...] = p_ref[...]

    for s in range(N_DEV - 1):
        sc_r = lax.rem(my_pos + 3 - s, N_DEV)
        rc_r = lax.rem(my_pos + 2 - s, N_DEV)
        sc_l = lax.rem(my_pos + 1 + s, N_DEV)
        rc_l = lax.rem(my_pos + 2 + s, N_DEV)
        rdma_r = pltpu.make_async_remote_copy(
            src_ref=w_ref.at[pl.ds(sc_r * CHUNK, CHUNK), 0:HALF],
            dst_ref=rs_r.at[s],
            send_sem=send_sems.at[s],
            recv_sem=recv_sems.at[s],
            device_id=(right,),
            device_id_type=pl.DeviceIdType.MESH,
        )
        rdma_l = pltpu.make_async_remote_copy(
            src_ref=w_ref.at[pl.ds(sc_l * CHUNK, CHUNK), HALF:D_MODEL],
            dst_ref=rs_l.at[s],
            send_sem=send_sems.at[6 + s],
            recv_sem=recv_sems.at[6 + s],
            device_id=(left,),
            device_id_type=pl.DeviceIdType.MESH,
        )
        rdma_r.start()
        rdma_l.start()
        rdma_r.wait()
        rdma_l.wait()
        w_ref[pl.ds(rc_r * CHUNK, CHUNK), 0:HALF] = (
            w_ref[pl.ds(rc_r * CHUNK, CHUNK), 0:HALF].astype(jnp.float32)
            + rs_r[s].astype(jnp.float32)
        ).astype(jnp.bfloat16)
        w_ref[pl.ds(rc_l * CHUNK, CHUNK), HALF:D_MODEL] = (
            w_ref[pl.ds(rc_l * CHUNK, CHUNK), HALF:D_MODEL].astype(jnp.float32)
            + rs_l[s].astype(jnp.float32)
        ).astype(jnp.bfloat16)

    for t in range(N_DEV - 1):
        gc_r = lax.rem(my_pos + N_DEV - t, N_DEV)
        gc_l = lax.rem(my_pos + t, N_DEV)
        rdma_r = pltpu.make_async_remote_copy(
            src_ref=w_ref.at[pl.ds(gc_r * CHUNK, CHUNK), 0:HALF],
            dst_ref=w_ref.at[pl.ds(gc_r * CHUNK, CHUNK), 0:HALF],
            send_sem=send_sems.at[N_DEV - 1 + t],
            recv_sem=recv_sems.at[N_DEV - 1 + t],
            device_id=(right,),
            device_id_type=pl.DeviceIdType.MESH,
        )
        rdma_l = pltpu.make_async_remote_copy(
            src_ref=w_ref.at[pl.ds(gc_l * CHUNK, CHUNK), HALF:D_MODEL],
            dst_ref=w_ref.at[pl.ds(gc_l * CHUNK, CHUNK), HALF:D_MODEL],
            send_sem=send_sems.at[6 + N_DEV - 1 + t],
            recv_sem=recv_sems.at[6 + N_DEV - 1 + t],
            device_id=(left,),
            device_id_type=pl.DeviceIdType.MESH,
        )
        rdma_r.start()
        rdma_l.start()
        rdma_r.wait()
        rdma_l.wait()
    out_ref[...] = w_ref[...].astype(jnp.float32)


def kernel(x, Wq, K_ext, V_ext, Wo):
    my = lax.axis_index("i")
    Wq_my = lax.dynamic_slice(Wq, (0, my * H_PER * DH), (D_MODEL, H_PER * DH))
    Wo_my = lax.dynamic_slice(Wo, (my * H_PER * DH, 0), (H_PER * DH, D_MODEL))

    partial = pl.pallas_call(
        _compute_body,
        out_shape=jax.ShapeDtypeStruct((SQ, D_MODEL), jnp.bfloat16),
        in_specs=[pl.BlockSpec(memory_space=pltpu.VMEM)] * 5,
        out_specs=pl.BlockSpec(memory_space=pltpu.VMEM),
        scratch_shapes=[
            pltpu.VMEM((SQ, D_MODEL), jnp.float32),
        ],
        compiler_params=pltpu.CompilerParams(
            vmem_limit_bytes=60 * 1024 * 1024,
        ),
    )(x[0], Wq_my, K_ext[0], V_ext[0], Wo_my)

    out = pl.pallas_call(
        _allreduce_body,
        out_shape=jax.ShapeDtypeStruct((SQ, D_MODEL), jnp.float32),
        in_specs=[pl.BlockSpec(memory_space=pltpu.VMEM)],
        out_specs=pl.BlockSpec(memory_space=pltpu.VMEM),
        scratch_shapes=[
            pltpu.VMEM((SQ, D_MODEL), jnp.bfloat16),
            pltpu.VMEM((N_DEV - 1, CHUNK, HALF), jnp.bfloat16),
            pltpu.VMEM((N_DEV - 1, CHUNK, HALF), jnp.bfloat16),
            pltpu.SemaphoreType.DMA((12,)),
            pltpu.SemaphoreType.DMA((12,)),
        ],
        compiler_params=pltpu.CompilerParams(
            collective_id=0,
            vmem_limit_bytes=60 * 1024 * 1024,
        ),
    )(partial)
    return out[None]


# device time: 122536 ns/iter; 1.4198x vs baseline; 1.0793x over previous
import jax
import jax.numpy as jnp
from jax import lax
from jax.experimental import pallas as pl
from jax.experimental.pallas import tpu as pltpu

N_DEV = 4
SQ = 2048
D_MODEL = 1024
H_PER = 8
DH = 128
BLK = 64
N_RES = 4
BLKS_PER_RES = SQ // BLK // N_RES
GROUP = BLKS_PER_RES * BLK
SCALE = 0.08838834764831843


def _compute_body(x_ref, wq_hbm, k_ref, v_ref, wo_hbm, out_ref,
                  acc_ref, wq_ref, wo_ref, dma_sems):
    my = lax.axis_index("i")
    cq = pltpu.make_async_copy(
        wq_hbm.at[:, pl.ds(my * H_PER * DH, H_PER * DH)],
        wq_ref, dma_sems.at[0])
    co = pltpu.make_async_copy(
        wo_hbm.at[pl.ds(my * H_PER * DH, H_PER * DH), :],
        wo_ref, dma_sems.at[1])
    cq.start()
    co.start()
    xv = x_ref[...].astype(jnp.bfloat16)
    acc_ref[...] = jnp.zeros((SQ, D_MODEL), jnp.float32)
    cq.wait()
    co.wait()
    for h in range(H_PER):
        wq_h = wq_ref[:, h * DH:(h + 1) * DH].astype(jnp.bfloat16)
        q_h = jnp.dot(xv, wq_h,
                      preferred_element_type=jnp.float32)
        k_h = k_ref[:, h, :].astype(jnp.bfloat16)
        v_h = v_ref[:, h, :].astype(jnp.bfloat16)
        qg = q_h.astype(jnp.bfloat16).reshape(BLKS_PER_RES, N_RES, BLK, DH)
        kg = k_h.reshape(BLKS_PER_RES, N_RES, BLK, DH)
        vg = v_h.reshape(BLKS_PER_RES, N_RES, BLK, DH)
        parts = []
        for r in range(N_RES):
            qr = qg[:, r].reshape(GROUP, DH)
            kr = kg[:, r].reshape(GROUP, DH)
            vr = vg[:, r].reshape(GROUP, DH)
            s = jnp.dot(qr, kr.T, preferred_element_type=jnp.float32) * SCALE
            e = jnp.exp(s)
            ssum = jnp.sum(e, axis=-1, keepdims=True)
            pv = jnp.dot(e.astype(jnp.bfloat16), vr,
                         preferred_element_type=jnp.float32)
            parts.append((pv / ssum).reshape(BLKS_PER_RES, BLK, DH))
        ctx_h = jnp.stack(parts, axis=1).reshape(SQ, DH).astype(jnp.bfloat16)
        wo_h = wo_ref[h * DH:(h + 1) * DH, :].astype(jnp.bfloat16)
        acc_ref[...] = acc_ref[...] + jnp.dot(
            ctx_h, wo_h, preferred_element_type=jnp.float32)
    out_ref[...] = acc_ref[...].astype(jnp.bfloat16)


CHUNK = SQ // N_DEV
HALF = D_MODEL // 2


def _allreduce_body(p_ref, out_ref, w_ref, rs_r, rs_l, send_sems, recv_sems):
    my_pos = lax.axis_index("i")
    left = lax.rem(my_pos + N_DEV - 1, N_DEV)
    right = lax.rem(my_pos + 1, N_DEV)

    barrier_sem = pltpu.get_barrier_semaphore()
    for nbr in (left, right):
        pl.semaphore_signal(
            barrier_sem, inc=1,
            device_id=(nbr,), device_id_type=pl.DeviceIdType.MESH,
        )
    pl.semaphore_wait(barrier_sem, 2)

    w_ref[...] = p_ref[...]

    for s in range(N_DEV - 1):
        sc_r = lax.rem(my_pos + 3 - s, N_DEV)
        rc_r = lax.rem(my_pos + 2 - s, N_DEV)
        sc_l = lax.rem(my_pos + 1 + s, N_DEV)
        rc_l = lax.rem(my_pos + 2 + s, N_DEV)
        rdma_r = pltpu.make_async_remote_copy(
            src_ref=w_ref.at[pl.ds(sc_r * CHUNK, CHUNK), 0:HALF],
            dst_ref=rs_r.at[s],
            send_sem=send_sems.at[s],
            recv_sem=recv_sems.at[s],
            device_id=(right,),
            device_id_type=pl.DeviceIdType.MESH,
        )
        rdma_l = pltpu.make_async_remote_copy(
            src_ref=w_ref.at[pl.ds(sc_l * CHUNK, CHUNK), HALF:D_MODEL],
            dst_ref=rs_l.at[s],
            send_sem=send_sems.at[6 + s],
            recv_sem=recv_sems.at[6 + s],
            device_id=(left,),
            device_id_type=pl.DeviceIdType.MESH,
        )
        rdma_r.start()
        rdma_l.start()
        rdma_r.wait()
        rdma_l.wait()
        w_ref[pl.ds(rc_r * CHUNK, CHUNK), 0:HALF] = (
            w_ref[pl.ds(rc_r * CHUNK, CHUNK), 0:HALF].astype(jnp.float32)
            + rs_r[s].astype(jnp.float32)
        ).astype(jnp.bfloat16)
        w_ref[pl.ds(rc_l * CHUNK, CHUNK), HALF:D_MODEL] = (
            w_ref[pl.ds(rc_l * CHUNK, CHUNK), HALF:D_MODEL].astype(jnp.float32)
            + rs_l[s].astype(jnp.float32)
        ).astype(jnp.bfloat16)

    for t in range(N_DEV - 1):
        gc_r = lax.rem(my_pos + N_DEV - t, N_DEV)
        gc_l = lax.rem(my_pos + t, N_DEV)
        rdma_r = pltpu.make_async_remote_copy(
            src_ref=w_ref.at[pl.ds(gc_r * CHUNK, CHUNK), 0:HALF],
            dst_ref=w_ref.at[pl.ds(gc_r * CHUNK, CHUNK), 0:HALF],
            send_sem=send_sems.at[N_DEV - 1 + t],
            recv_sem=recv_sems.at[N_DEV - 1 + t],
            device_id=(right,),
            device_id_type=pl.DeviceIdType.MESH,
        )
        rdma_l = pltpu.make_async_remote_copy(
            src_ref=w_ref.at[pl.ds(gc_l * CHUNK, CHUNK), HALF:D_MODEL],
            dst_ref=w_ref.at[pl.ds(gc_l * CHUNK, CHUNK), HALF:D_MODEL],
            send_sem=send_sems.at[6 + N_DEV - 1 + t],
            recv_sem=recv_sems.at[6 + N_DEV - 1 + t],
            device_id=(left,),
            device_id_type=pl.DeviceIdType.MESH,
        )
        rdma_r.start()
        rdma_l.start()
        rdma_r.wait()
        rdma_l.wait()
    out_ref[...] = w_ref[...].astype(jnp.float32)


def kernel(x, Wq, K_ext, V_ext, Wo):
    partial = pl.pallas_call(
        _compute_body,
        out_shape=jax.ShapeDtypeStruct((SQ, D_MODEL), jnp.bfloat16),
        in_specs=[
            pl.BlockSpec(memory_space=pltpu.VMEM),
            pl.BlockSpec(memory_space=pltpu.MemorySpace.HBM),
            pl.BlockSpec(memory_space=pltpu.VMEM),
            pl.BlockSpec(memory_space=pltpu.VMEM),
            pl.BlockSpec(memory_space=pltpu.MemorySpace.HBM),
        ],
        out_specs=pl.BlockSpec(memory_space=pltpu.VMEM),
        scratch_shapes=[
            pltpu.VMEM((SQ, D_MODEL), jnp.float32),
            pltpu.VMEM((D_MODEL, H_PER * DH), jnp.float32),
            pltpu.VMEM((H_PER * DH, D_MODEL), jnp.float32),
            pltpu.SemaphoreType.DMA((2,)),
        ],
        compiler_params=pltpu.CompilerParams(
            vmem_limit_bytes=60 * 1024 * 1024,
        ),
    )(x[0], Wq, K_ext[0], V_ext[0], Wo)

    out = pl.pallas_call(
        _allreduce_body,
        out_shape=jax.ShapeDtypeStruct((SQ, D_MODEL), jnp.float32),
        in_specs=[pl.BlockSpec(memory_space=pltpu.VMEM)],
        out_specs=pl.BlockSpec(memory_space=pltpu.VMEM),
        scratch_shapes=[
            pltpu.VMEM((SQ, D_MODEL), jnp.bfloat16),
            pltpu.VMEM((N_DEV - 1, CHUNK, HALF), jnp.bfloat16),
            pltpu.VMEM((N_DEV - 1, CHUNK, HALF), jnp.bfloat16),
            pltpu.SemaphoreType.DMA((12,)),
            pltpu.SemaphoreType.DMA((12,)),
        ],
        compiler_params=pltpu.CompilerParams(
            collective_id=0,
            vmem_limit_bytes=60 * 1024 * 1024,
        ),
    )(partial)
    return out[None]


# device time: 108433 ns/iter; 1.6045x vs baseline; 1.1301x over previous
import jax
import jax.numpy as jnp
from jax import lax
from jax.experimental import pallas as pl
from jax.experimental.pallas import tpu as pltpu

N_DEV = 4
SQ = 2048
D_MODEL = 1024
H_PER = 8
DH = 128
BLK = 64
N_RES = 4
BLKS_PER_RES = SQ // BLK // N_RES
GROUP = BLKS_PER_RES * BLK
SCALE = 0.08838834764831843


def _compute_body(x_ref, wq_hbm, k_ref, v_ref, wo_hbm, out_ref,
                  ctx_ref, wq_ref, wo_ref, dma_sems):
    my = lax.axis_index("i")
    cq = pltpu.make_async_copy(
        wq_hbm.at[:, pl.ds(my * H_PER * DH, H_PER * DH)],
        wq_ref, dma_sems.at[0])
    co = pltpu.make_async_copy(
        wo_hbm.at[pl.ds(my * H_PER * DH, H_PER * DH), :],
        wo_ref, dma_sems.at[1])
    cq.start()
    co.start()
    xv = x_ref[...].astype(jnp.bfloat16)
    cq.wait()
    co.wait()
    q_all = jnp.dot(xv, wq_ref[...].astype(jnp.bfloat16),
                    preferred_element_type=jnp.float32).astype(jnp.bfloat16)
    for h in range(H_PER):
        q_h = q_all[:, h * DH:(h + 1) * DH]
        k_h = k_ref[:, h, :].astype(jnp.bfloat16)
        v_h = v_ref[:, h, :].astype(jnp.bfloat16)
        qg = q_h.reshape(BLKS_PER_RES, N_RES, BLK, DH)
        kg = k_h.reshape(BLKS_PER_RES, N_RES, BLK, DH)
        vg = v_h.reshape(BLKS_PER_RES, N_RES, BLK, DH)
        parts = []
        for r in range(N_RES):
            qr = qg[:, r].reshape(GROUP, DH)
            kr = kg[:, r].reshape(GROUP, DH)
            vr = vg[:, r].reshape(GROUP, DH)
            s = jnp.dot(qr, kr.T, preferred_element_type=jnp.float32) * SCALE
            e = jnp.exp(s)
            ssum = jnp.sum(e, axis=-1, keepdims=True)
            pv = jnp.dot(e.astype(jnp.bfloat16), vr,
                         preferred_element_type=jnp.float32)
            parts.append((pv / ssum).reshape(BLKS_PER_RES, BLK, DH))
        ctx_ref[:, h * DH:(h + 1) * DH] = (
            jnp.stack(parts, axis=1).reshape(SQ, DH).astype(jnp.bfloat16))
    out_ref[...] = jnp.dot(
        ctx_ref[...], wo_ref[...].astype(jnp.bfloat16),
        preferred_element_type=jnp.float32).astype(jnp.bfloat16)


CHUNK = SQ // N_DEV
HALF = D_MODEL // 2


def _allreduce_body(p_ref, out_ref, w_ref, rs_r, rs_l, send_sems, recv_sems):
    my_pos = lax.axis_index("i")
    left = lax.rem(my_pos + N_DEV - 1, N_DEV)
    right = lax.rem(my_pos + 1, N_DEV)

    barrier_sem = pltpu.get_barrier_semaphore()
    for nbr in (left, right):
        pl.semaphore_signal(
            barrier_sem, inc=1,
            device_id=(nbr,), device_id_type=pl.DeviceIdType.MESH,
        )
    pl.semaphore_wait(barrier_sem, 2)

    w_ref[...] = p_ref[...]

    for s in range(N_DEV - 1):
        sc_r = lax.rem(my_pos + 3 - s, N_DEV)
        rc_r = lax.rem(my_pos + 2 - s, N_DEV)
        sc_l = lax.rem(my_pos + 1 + s, N_DEV)
        rc_l = lax.rem(my_pos + 2 + s, N_DEV)
        rdma_r = pltpu.make_async_remote_copy(
            src_ref=w_ref.at[pl.ds(sc_r * CHUNK, CHUNK), 0:HALF],
            dst_ref=rs_r.at[s],
            send_sem=send_sems.at[s],
            recv_sem=recv_sems.at[s],
            device_id=(right,),
            device_id_type=pl.DeviceIdType.MESH,
        )
        rdma_l = pltpu.make_async_remote_copy(
            src_ref=w_ref.at[pl.ds(sc_l * CHUNK, CHUNK), HALF:D_MODEL],
            dst_ref=rs_l.at[s],
            send_sem=send_sems.at[6 + s],
            recv_sem=recv_sems.at[6 + s],
            device_id=(left,),
            device_id_type=pl.DeviceIdType.MESH,
        )
        rdma_r.start()
        rdma_l.start()
        rdma_r.wait()
        w_ref[pl.ds(rc_r * CHUNK, CHUNK), 0:HALF] = (
            w_ref[pl.ds(rc_r * CHUNK, CHUNK), 0:HALF].astype(jnp.float32)
            + rs_r[s].astype(jnp.float32)
        ).astype(jnp.bfloat16)
        rdma_l.wait()
        w_ref[pl.ds(rc_l * CHUNK, CHUNK), HALF:D_MODEL] = (
            w_ref[pl.ds(rc_l * CHUNK, CHUNK), HALF:D_MODEL].astype(jnp.float32)
            + rs_l[s].astype(jnp.float32)
        ).astype(jnp.bfloat16)

    for t in range(N_DEV - 1):
        gc_r = lax.rem(my_pos + N_DEV - t, N_DEV)
        gc_l = lax.rem(my_pos + t, N_DEV)
        rdma_r = pltpu.make_async_remote_copy(
            src_ref=w_ref.at[pl.ds(gc_r * CHUNK, CHUNK), 0:HALF],
            dst_ref=w_ref.at[pl.ds(gc_r * CHUNK, CHUNK), 0:HALF],
            send_sem=send_sems.at[N_DEV - 1 + t],
            recv_sem=recv_sems.at[N_DEV - 1 + t],
            device_id=(right,),
            device_id_type=pl.DeviceIdType.MESH,
        )
        rdma_l = pltpu.make_async_remote_copy(
            src_ref=w_ref.at[pl.ds(gc_l * CHUNK, CHUNK), HALF:D_MODEL],
            dst_ref=w_ref.at[pl.ds(gc_l * CHUNK, CHUNK), HALF:D_MODEL],
            send_sem=send_sems.at[6 + N_DEV - 1 + t],
            recv_sem=recv_sems.at[6 + N_DEV - 1 + t],
            device_id=(left,),
            device_id_type=pl.DeviceIdType.MESH,
        )
        rdma_r.start()
        rdma_l.start()
        rdma_r.wait()
        rdma_l.wait()
    out_ref[...] = w_ref[...].astype(jnp.float32)


def kernel(x, Wq, K_ext, V_ext, Wo):
    partial = pl.pallas_call(
        _compute_body,
        out_shape=jax.ShapeDtypeStruct((SQ, D_MODEL), jnp.bfloat16),
        in_specs=[
            pl.BlockSpec(memory_space=pltpu.VMEM),
            pl.BlockSpec(memory_space=pltpu.MemorySpace.HBM),
            pl.BlockSpec(memory_space=pltpu.VMEM),
            pl.BlockSpec(memory_space=pltpu.VMEM),
            pl.BlockSpec(memory_space=pltpu.MemorySpace.HBM),
        ],
        out_specs=pl.BlockSpec(memory_space=pltpu.VMEM),
        scratch_shapes=[
            pltpu.VMEM((SQ, D_MODEL), jnp.bfloat16),
            pltpu.VMEM((D_MODEL, H_PER * DH), jnp.float32),
            pltpu.VMEM((H_PER * DH, D_MODEL), jnp.float32),
            pltpu.SemaphoreType.DMA((2,)),
        ],
        compiler_params=pltpu.CompilerParams(
            vmem_limit_bytes=60 * 1024 * 1024,
        ),
    )(x[0], Wq, K_ext[0], V_ext[0], Wo)

    out = pl.pallas_call(
        _allreduce_body,
        out_shape=jax.ShapeDtypeStruct((SQ, D_MODEL), jnp.float32),
        in_specs=[pl.BlockSpec(memory_space=pltpu.VMEM)],
        out_specs=pl.BlockSpec(memory_space=pltpu.VMEM),
        scratch_shapes=[
            pltpu.VMEM((SQ, D_MODEL), jnp.bfloat16),
            pltpu.VMEM((N_DEV - 1, CHUNK, HALF), jnp.bfloat16),
            pltpu.VMEM((N_DEV - 1, CHUNK, HALF), jnp.bfloat16),
            pltpu.SemaphoreType.DMA((12,)),
            pltpu.SemaphoreType.DMA((12,)),
        ],
        compiler_params=pltpu.CompilerParams(
            collective_id=0,
            vmem_limit_bytes=60 * 1024 * 1024,
        ),
    )(partial)
    return out[None]


# device time: 100550 ns/iter; 1.7302x vs baseline; 1.0784x over previous
import jax
import jax.numpy as jnp
from jax import lax
from jax.experimental import pallas as pl
from jax.experimental.pallas import tpu as pltpu

N_DEV = 4
SQ = 2048
D_MODEL = 1024
H_PER = 8
DH = 128
BLK = 64
N_RES = 4
BLKS_PER_RES = SQ // BLK // N_RES
GROUP = BLKS_PER_RES * BLK
SCALE = 0.08838834764831843


def _compute_body(x_ref, wq_hbm, k_ref, v_ref, wo_hbm, out_ref,
                  ctx_ref, wq_ref, wo_ref, dma_sems):
    my = lax.axis_index("i")
    cq = pltpu.make_async_copy(
        wq_hbm.at[:, pl.ds(my * H_PER * DH, H_PER * DH)],
        wq_ref, dma_sems.at[0])
    co = pltpu.make_async_copy(
        wo_hbm.at[pl.ds(my * H_PER * DH, H_PER * DH), :],
        wo_ref, dma_sems.at[1])
    cq.start()
    co.start()
    xv = x_ref[...].astype(jnp.bfloat16)
    cq.wait()
    q_all = jnp.dot(xv, wq_ref[...].astype(jnp.bfloat16),
                    preferred_element_type=jnp.float32).astype(jnp.bfloat16)
    for h in range(H_PER):
        q_h = q_all[:, h * DH:(h + 1) * DH]
        k_h = k_ref[:, h, :].astype(jnp.bfloat16)
        v_h = v_ref[:, h, :].astype(jnp.bfloat16)
        qg = q_h.reshape(BLKS_PER_RES, N_RES, BLK, DH)
        kg = k_h.reshape(BLKS_PER_RES, N_RES, BLK, DH)
        vg = v_h.reshape(BLKS_PER_RES, N_RES, BLK, DH)
        parts = []
        for r in range(N_RES):
            qr = qg[:, r].reshape(GROUP, DH)
            kr = kg[:, r].reshape(GROUP, DH)
            vr = vg[:, r].reshape(GROUP, DH)
            s = jnp.dot(qr, kr.T, preferred_element_type=jnp.float32) * SCALE
            e = jnp.exp(s)
            ssum = jnp.sum(e, axis=-1, keepdims=True)
            pv = jnp.dot(e.astype(jnp.bfloat16), vr,
                         preferred_element_type=jnp.float32)
            parts.append((pv / ssum).reshape(BLKS_PER_RES, BLK, DH))
        ctx_ref[:, h * DH:(h + 1) * DH] = (
            jnp.stack(parts, axis=1).reshape(SQ, DH).astype(jnp.bfloat16))
    co.wait()
    out_ref[...] = jnp.dot(
        ctx_ref[...], wo_ref[...].astype(jnp.bfloat16),
        preferred_element_type=jnp.float32).astype(jnp.bfloat16)


CHUNK = SQ // N_DEV
HALF = D_MODEL // 2


def _allreduce_body(p_ref, out_ref, w_ref, rs_bufs, send_sems, recv_sems):
    my_pos = lax.axis_index("i")
    left = lax.rem(my_pos + N_DEV - 1, N_DEV)
    right = lax.rem(my_pos + 1, N_DEV)

    barrier_sem = pltpu.get_barrier_semaphore()
    for nbr in (left, right):
        pl.semaphore_signal(
            barrier_sem, inc=1,
            device_id=(nbr,), device_id_type=pl.DeviceIdType.MESH,
        )
    pl.semaphore_wait(barrier_sem, 2)

    w_ref[...] = p_ref[...]

    QTR = D_MODEL // 4
    COL0 = (0, QTR, 2 * QTR, 3 * QTR)

    def rs_chunk(g, s, recv):
        if g < 2:
            return lax.rem(my_pos + (2 if recv else 3) - s, N_DEV)
        return lax.rem(my_pos + (2 if recv else 1) + s, N_DEV)

    def make_rs(g, s):
        sc = rs_chunk(g, s, recv=False)
        return pltpu.make_async_remote_copy(
            src_ref=w_ref.at[pl.ds(sc * CHUNK, CHUNK),
                             COL0[g]:COL0[g] + QTR],
            dst_ref=rs_bufs.at[g, s],
            send_sem=send_sems.at[g * 6 + s],
            recv_sem=recv_sems.at[g * 6 + s],
            device_id=(right if g < 2 else left,),
            device_id_type=pl.DeviceIdType.MESH,
        )

    def acc_rs(g, s):
        rc = rs_chunk(g, s, recv=True)
        w_ref[pl.ds(rc * CHUNK, CHUNK), COL0[g]:COL0[g] + QTR] = (
            w_ref[pl.ds(rc * CHUNK, CHUNK), COL0[g]:COL0[g] + QTR]
            .astype(jnp.float32) + rs_bufs[g, s].astype(jnp.float32)
        ).astype(jnp.bfloat16)

    def make_ag(g, t):
        if g < 2:
            gc = lax.rem(my_pos + N_DEV - t, N_DEV)
        else:
            gc = lax.rem(my_pos + t, N_DEV)
        ref = w_ref.at[pl.ds(gc * CHUNK, CHUNK), COL0[g]:COL0[g] + QTR]
        return pltpu.make_async_remote_copy(
            src_ref=ref, dst_ref=ref,
            send_sem=send_sems.at[g * 6 + 3 + t],
            recv_sem=recv_sems.at[g * 6 + 3 + t],
            device_id=(right if g < 2 else left,),
            device_id_type=pl.DeviceIdType.MESH,
        )

    cur = [make_rs(g, 0) for g in range(4)]
    for r in cur:
        r.start()
    for s in range(N_DEV - 1):
        nxt = []
        for g in range(4):
            cur[g].wait()
            acc_rs(g, s)
            r = make_rs(g, s + 1) if s < N_DEV - 2 else make_ag(g, 0)
            r.start()
            nxt.append(r)
        cur = nxt

    for t in range(N_DEV - 1):
        nxt = []
        for g in range(4):
            cur[g].wait()
            if t < N_DEV - 2:
                r = make_ag(g, t + 1)
                r.start()
                nxt.append(r)
        cur = nxt
    out_ref[...] = w_ref[...].astype(jnp.float32)


def kernel(x, Wq, K_ext, V_ext, Wo):
    partial = pl.pallas_call(
        _compute_body,
        out_shape=jax.ShapeDtypeStruct((SQ, D_MODEL), jnp.bfloat16),
        in_specs=[
            pl.BlockSpec(memory_space=pltpu.VMEM),
            pl.BlockSpec(memory_space=pltpu.MemorySpace.HBM),
            pl.BlockSpec(memory_space=pltpu.VMEM),
            pl.BlockSpec(memory_space=pltpu.VMEM),
            pl.BlockSpec(memory_space=pltpu.MemorySpace.HBM),
        ],
        out_specs=pl.BlockSpec(memory_space=pltpu.VMEM),
        scratch_shapes=[
            pltpu.VMEM((SQ, D_MODEL), jnp.bfloat16),
            pltpu.VMEM((D_MODEL, H_PER * DH), jnp.float32),
            pltpu.VMEM((H_PER * DH, D_MODEL), jnp.float32),
            pltpu.SemaphoreType.DMA((2,)),
        ],
        compiler_params=pltpu.CompilerParams(
            vmem_limit_bytes=60 * 1024 * 1024,
        ),
    )(x[0], Wq, K_ext[0], V_ext[0], Wo)

    out = pl.pallas_call(
        _allreduce_body,
        out_shape=jax.ShapeDtypeStruct((SQ, D_MODEL), jnp.float32),
        in_specs=[pl.BlockSpec(memory_space=pltpu.VMEM)],
        out_specs=pl.BlockSpec(memory_space=pltpu.VMEM),
        scratch_shapes=[
            pltpu.VMEM((SQ, D_MODEL), jnp.bfloat16),
            pltpu.VMEM((4, N_DEV - 1, CHUNK, D_MODEL // 4),
                       jnp.bfloat16),
            pltpu.SemaphoreType.DMA((24,)),
            pltpu.SemaphoreType.DMA((24,)),
        ],
        compiler_params=pltpu.CompilerParams(
            collective_id=0,
            vmem_limit_bytes=60 * 1024 * 1024,
        ),
    )(partial)
    return out[None]
